# baseline (speedup 1.0000x reference)
import functools

import jax
import jax.numpy as jnp
from jax import lax
from jax.experimental import pallas as pl
from jax.experimental.pallas import tpu as pltpu
from jax.experimental.pallas import tpu_sc as plsc

T = 2048
D = 1024
H = 2048
E = 8
RB = 128
NB = (2 * T + E * RB) // RB
PAD = NB * RB
GW = 128

NC = 2
NS = 16
NW = NC * NS
TPW = T // NW


def _router_body(p_ref, probs_ref, rows0_ref, rows1_ref,
                 gw0_ref, gw1_ref, bm_ref, par_ref, first_ref, fire_ref,
                 nexte_ref):
    p = p_ref[...]
    probs_ref[...] = p

    r8 = lax.broadcasted_iota(jnp.int32, (E, E), 0)
    c8 = lax.broadcasted_iota(jnp.int32, (E, E), 1)
    tri8 = (r8 <= c8).astype(jnp.float32)
    p1 = jnp.max(p, axis=-1, keepdims=True)
    oh1 = (p == p1)
    cum1 = jnp.dot(oh1.astype(jnp.float32), tri8,
                   preferred_element_type=jnp.float32)
    first1 = oh1 & (cum1 == 1.0)
    pm = jnp.where(first1, -1.0, p)
    p2 = jnp.max(pm, axis=-1, keepdims=True)
    oh2 = (pm == p2)
    cum2 = jnp.dot(oh2.astype(jnp.float32), tri8,
                   preferred_element_type=jnp.float32)
    first2 = oh2 & (cum2 == 1.0)

    s1 = first1.astype(jnp.float32)
    s2 = first2.astype(jnp.float32)
    sel = s1 + s2

    rT = lax.broadcasted_iota(jnp.int32, (T, T), 0)
    cT = lax.broadcasted_iota(jnp.int32, (T, T), 1)
    ltri = (rT > cT).astype(jnp.bfloat16)
    rank = jnp.dot(ltri, sel.astype(jnp.bfloat16),
                   preferred_element_type=jnp.float32)

    counts = jnp.sum(sel, axis=0, keepdims=True)
    pc = jnp.ceil(counts / RB) * RB
    m8 = (r8 < c8).astype(jnp.float32)
    starts = jnp.dot(pc, m8, preferred_element_type=jnp.float32)

    slot = starts + rank
    rows0 = jnp.sum(jnp.where(first1, slot, 0.0), axis=-1, keepdims=True)
    rows1 = jnp.sum(jnp.where(first2, slot, 0.0), axis=-1, keepdims=True)
    rows0_ref[...] = rows0.astype(jnp.int32)
    rows1_ref[...] = rows1.astype(jnp.int32)

    p0 = jnp.sum(jnp.where(first1, p, 0.0), axis=-1, keepdims=True)
    p1v = jnp.sum(jnp.where(first2, p, 0.0), axis=-1, keepdims=True)
    gw0_ref[...] = jnp.broadcast_to(p0, (T, GW))
    gw1_ref[...] = jnp.broadcast_to(p1v, (T, GW))

    sb = starts / RB
    ib = lax.broadcasted_iota(jnp.int32, (NB, E), 0).astype(jnp.float32)
    cmp = (ib >= sb).astype(jnp.float32)
    bm = jnp.sum(cmp, axis=-1, keepdims=True) - 1.0
    bm_ref[...] = bm.astype(jnp.int32)

    iN = lax.broadcasted_iota(jnp.int32, (NB, NB), 0)
    jN = lax.broadcasted_iota(jnp.int32, (NB, NB), 1)
    shift = ((jN == iN - 1)).astype(jnp.float32)
    bmprev = jnp.dot(shift, bm, preferred_element_type=jnp.float32)
    i0 = (iN[:, 0:1] == 0)
    first = jnp.where(i0 | (bm != bmprev), 1.0, 0.0)
    triN = (jN <= iN).astype(jnp.float32)
    runidx = jnp.dot(triN, first, preferred_element_type=jnp.float32) - 1.0
    par = runidx - 2.0 * jnp.floor(runidx * 0.5)
    eyeN = (iN == jN).astype(jnp.float32)
    firstrow = jnp.dot(jnp.ones((1, NB), jnp.float32), first * eyeN,
                       preferred_element_type=jnp.float32)
    cand = jnp.where((firstrow > 0.0) & (jN > iN),
                     jN.astype(jnp.float32), float(NB))
    nb = jnp.min(cand, axis=-1, keepdims=True)
    fire = jnp.where((first > 0.0) & (nb < float(NB)), 1.0, 0.0)
    ohnb = (jN.astype(jnp.float32) == nb).astype(jnp.float32)
    nexte = jnp.dot(ohnb, bm, preferred_element_type=jnp.float32)
    par_ref[...] = par.astype(jnp.int32)
    first_ref[...] = first.astype(jnp.int32)
    fire_ref[...] = fire.astype(jnp.int32)
    nexte_ref[...] = nexte.astype(jnp.int32)


def _stage_a(probs):
    return pl.pallas_call(
        _router_body,
        in_specs=[
            pl.BlockSpec((T, E), lambda: (0, 0)),
        ],
        out_specs=[
            pl.BlockSpec((T, E), lambda: (0, 0)),
            pl.BlockSpec((T, 1), lambda: (0, 0)),
            pl.BlockSpec((T, 1), lambda: (0, 0)),
            pl.BlockSpec((T, GW), lambda: (0, 0)),
            pl.BlockSpec((T, GW), lambda: (0, 0)),
            pl.BlockSpec((NB, 1), lambda: (0, 0)),
            pl.BlockSpec((NB, 1), lambda: (0, 0)),
            pl.BlockSpec((NB, 1), lambda: (0, 0)),
            pl.BlockSpec((NB, 1), lambda: (0, 0)),
            pl.BlockSpec((NB, 1), lambda: (0, 0)),
        ],
        out_shape=[
            jax.ShapeDtypeStruct((T, E), jnp.float32),
            jax.ShapeDtypeStruct((T, 1), jnp.int32),
            jax.ShapeDtypeStruct((T, 1), jnp.int32),
            jax.ShapeDtypeStruct((T, GW), jnp.float32),
            jax.ShapeDtypeStruct((T, GW), jnp.float32),
            jax.ShapeDtypeStruct((NB, 1), jnp.int32),
            jax.ShapeDtypeStruct((NB, 1), jnp.int32),
            jax.ShapeDtypeStruct((NB, 1), jnp.int32),
            jax.ShapeDtypeStruct((NB, 1), jnp.int32),
            jax.ShapeDtypeStruct((NB, 1), jnp.int32),
        ],
    )(probs)


def _dispatch_body(x_hbm, rows0_hbm, rows1_hbm, gw0_hbm, gw1_hbm,
                   xs_hbm, g_hbm, idx0_v, idx1_v, xchunk, g0chunk, g1chunk,
                   sem):
    wid = lax.axis_index("s") * NC + lax.axis_index("c")
    base = wid * TPW
    pltpu.sync_copy(rows0_hbm.at[pl.ds(base, TPW)], idx0_v)
    pltpu.sync_copy(rows1_hbm.at[pl.ds(base, TPW)], idx1_v)
    pltpu.sync_copy(x_hbm.at[pl.ds(base, TPW), :], xchunk)
    pltpu.sync_copy(gw0_hbm.at[pl.ds(base, TPW), :], g0chunk)
    pltpu.sync_copy(gw1_hbm.at[pl.ds(base, TPW), :], g1chunk)
    c0 = pltpu.async_copy(xchunk, xs_hbm.at[idx0_v], sem)
    c1 = pltpu.async_copy(xchunk, xs_hbm.at[idx1_v], sem)
    c2 = pltpu.async_copy(g0chunk, g_hbm.at[idx0_v], sem)
    c3 = pltpu.async_copy(g1chunk, g_hbm.at[idx1_v], sem)
    c0.wait()
    c1.wait()
    c2.wait()
    c3.wait()


def _stage_b(x, rows0, rows1, gw0, gw1):
    mesh = plsc.VectorSubcoreMesh(core_axis_name="c", subcore_axis_name="s",
                                  num_cores=NC, num_subcores=NS)
    f = pl.kernel(
        _dispatch_body,
        out_type=[
            jax.ShapeDtypeStruct((PAD, D), jnp.float32),
            jax.ShapeDtypeStruct((PAD, GW), jnp.float32),
        ],
        mesh=mesh,
        scratch_types=[
            pltpu.VMEM((TPW,), jnp.int32),
            pltpu.VMEM((TPW,), jnp.int32),
            pltpu.VMEM((TPW, D), jnp.float32),
            pltpu.VMEM((TPW, GW), jnp.float32),
            pltpu.VMEM((TPW, GW), jnp.float32),
            pltpu.SemaphoreType.DMA,
        ],
    )
    return f(x, rows0, rows1, gw0, gw1)


def _ffn_body(bm_s, par_s, first_s, fire_s, nexte_s,
              xs_ref, g_ref, w1_any, w2_any, b1_ref, b2_ref, ys_ref,
              w1s, w2s, w1c, w2c, sems):
    i = pl.program_id(0)
    e = bm_s[i]
    par = par_s[i]
    first = first_s[i]
    fire = fire_s[i]
    nxt = nexte_s[i]

    @pl.when(i == 0)
    def _prime():
        pltpu.make_async_copy(w1_any.at[pl.ds(e, 1)],
                              w1s.at[pl.ds(0, 1)], sems.at[0]).start()
        pltpu.make_async_copy(w2_any.at[pl.ds(e, 1)],
                              w2s.at[pl.ds(0, 1)], sems.at[0]).start()

    @pl.when(first == 1)
    def _wait():
        pltpu.make_async_copy(w1_any.at[pl.ds(e, 1)],
                              w1s.at[pl.ds(par, 1)], sems.at[par]).wait()
        pltpu.make_async_copy(w2_any.at[pl.ds(e, 1)],
                              w2s.at[pl.ds(par, 1)], sems.at[par]).wait()
        w1c[...] = w1s[par].astype(jnp.bfloat16)
        w2c[...] = w2s[par].astype(jnp.bfloat16)

    @pl.when(fire == 1)
    def _fire():
        pltpu.make_async_copy(w1_any.at[pl.ds(nxt, 1)],
                              w1s.at[pl.ds(1 - par, 1)],
                              sems.at[1 - par]).start()
        pltpu.make_async_copy(w2_any.at[pl.ds(nxt, 1)],
                              w2s.at[pl.ds(1 - par, 1)],
                              sems.at[1 - par]).start()

    x = xs_ref[...].astype(jnp.bfloat16)
    h = jnp.dot(x, w1c[...], preferred_element_type=jnp.float32)
    h = jnp.maximum(h + b1_ref[0], 0.0)
    y = jnp.dot(h.astype(jnp.bfloat16), w2c[...],
                preferred_element_type=jnp.float32)
    ys_ref[...] = (y + b2_ref[0]) * g_ref[:, 0:1]


def _stage_c(bm, par, first, fire, nexte, xs, g, W1, b1, W2, b2):
    grid_spec = pltpu.PrefetchScalarGridSpec(
        num_scalar_prefetch=5,
        grid=(NB,),
        in_specs=[
            pl.BlockSpec((RB, D), lambda i, *s: (i, 0)),
            pl.BlockSpec((RB, GW), lambda i, *s: (i, 0)),
            pl.BlockSpec(memory_space=pl.ANY),
            pl.BlockSpec(memory_space=pl.ANY),
            pl.BlockSpec((1, 1, H), lambda i, bm, *s: (bm[i], 0, 0)),
            pl.BlockSpec((1, 1, D), lambda i, bm, *s: (bm[i], 0, 0)),
        ],
        out_specs=pl.BlockSpec((RB, D), lambda i, *s: (i, 0)),
        scratch_shapes=[
            pltpu.VMEM((2, D, H), jnp.float32),
            pltpu.VMEM((2, H, D), jnp.float32),
            pltpu.VMEM((D, H), jnp.bfloat16),
            pltpu.VMEM((H, D), jnp.bfloat16),
            pltpu.SemaphoreType.DMA((2,)),
        ],
    )
    return pl.pallas_call(
        _ffn_body,
        grid_spec=grid_spec,
        out_shape=jax.ShapeDtypeStruct((PAD, D), jnp.float32),
    )(bm, par, first, fire, nexte, xs, g, W1, W2,
      b1.reshape(E, 1, H), b2.reshape(E, 1, D))


CHUNK = 16
NSUB = TPW // CHUNK


def _combine_body(ys_hbm, rows0_hbm, rows1_hbm, out_hbm,
                  idx0_a, idx1_a, buf0_a, buf1_a, idx0_b, idx1_b, buf0_b,
                  buf1_b, sem_a, sem_b):
    wid = lax.axis_index("s") * NC + lax.axis_index("c")
    bufs = [(idx0_a, idx1_a, buf0_a, buf1_a, sem_a),
            (idx0_b, idx1_b, buf0_b, buf1_b, sem_b)]

    def fire(k):
        idx0, idx1, b0, b1, sem = bufs[k % 2]
        base = wid * TPW + k * CHUNK
        pltpu.sync_copy(rows0_hbm.at[pl.ds(base, CHUNK)], idx0)
        pltpu.sync_copy(rows1_hbm.at[pl.ds(base, CHUNK)], idx1)
        return (pltpu.async_copy(ys_hbm.at[idx0], b0, sem),
                pltpu.async_copy(ys_hbm.at[idx1], b1, sem))

    def drain_compute(k, cps):
        idx0, idx1, b0, b1, sem = bufs[k % 2]
        base = wid * TPW + k * CHUNK
        cps[0].wait()
        cps[1].wait()

        def add_row(r, _):
            for c in range(D // 16):
                b0[r, pl.ds(c * 16, 16)] = (b0[r, pl.ds(c * 16, 16)]
                                            + b1[r, pl.ds(c * 16, 16)])
            return 0

        lax.fori_loop(0, CHUNK, add_row, 0)
        pltpu.sync_copy(b0, out_hbm.at[pl.ds(base, CHUNK), :])

    cps = fire(0)
    for k in range(NSUB):
        nxt = fire(k + 1) if k + 1 < NSUB else None
        drain_compute(k, cps)
        cps = nxt


def _stage_d(ys, rows0, rows1):
    mesh = plsc.VectorSubcoreMesh(core_axis_name="c", subcore_axis_name="s",
                                  num_cores=NC, num_subcores=NS)
    f = pl.kernel(
        _combine_body,
        out_type=jax.ShapeDtypeStruct((T, D), jnp.float32),
        mesh=mesh,
        scratch_types=[
            pltpu.VMEM((CHUNK,), jnp.int32),
            pltpu.VMEM((CHUNK,), jnp.int32),
            pltpu.VMEM((CHUNK, D), jnp.float32),
            pltpu.VMEM((CHUNK, D), jnp.float32),
            pltpu.VMEM((CHUNK,), jnp.int32),
            pltpu.VMEM((CHUNK,), jnp.int32),
            pltpu.VMEM((CHUNK, D), jnp.float32),
            pltpu.VMEM((CHUNK, D), jnp.float32),
            pltpu.SemaphoreType.DMA,
            pltpu.SemaphoreType.DMA,
        ],
    )
    return f(ys, rows0, rows1)


@jax.jit
def kernel(embedding, Wg, bg, R, W1, b1, W2, b2):
    x = embedding.reshape(T, D)

    state = jnp.zeros(embedding.shape[:-1] + (E,), dtype=embedding.dtype).at[..., 0].set(1.0)
    tb = jnp.einsum('ef,bsf->bse', R, state)
    logits = jnp.einsum('bsd,de->bse', embedding, Wg) + bg + tb
    probs_in = jax.nn.softmax(logits, axis=-1).reshape(T, E)

    (probs, rows0, rows1, gw0, gw1, bm, par, first, fire,
     nexte) = _stage_a(probs_in)
    rows0 = rows0.reshape(T)
    rows1 = rows1.reshape(T)
    xs, g = _stage_b(x, rows0, rows1, gw0, gw1)
    ys = _stage_c(bm.reshape(NB), par.reshape(NB), first.reshape(NB),
                  fire.reshape(NB), nexte.reshape(NB), xs, g, W1, b1, W2, b2)
    out = _stage_d(ys, rows0, rows1)
    return out.reshape(1, T, D), probs.reshape(1, T, E)


# speedup vs baseline: 1.0670x; 1.0670x over previous
import functools

import jax
import jax.numpy as jnp
from jax import lax
from jax.experimental import pallas as pl
from jax.experimental.pallas import tpu as pltpu
from jax.experimental.pallas import tpu_sc as plsc

T = 2048
D = 1024
H = 2048
E = 8
RB = 128
NB = (2 * T + E * RB) // RB
PAD = NB * RB
GW = 128

NC = 2
NS = 16
NW = NC * NS
TPW = T // NW


def _router_body(p_ref, probs_ref, rows0_ref, rows1_ref,
                 gw0_ref, gw1_ref, bm_ref, par_ref, first_ref, fire_ref,
                 nexte_ref):
    p = p_ref[...]
    probs_ref[...] = p

    r8 = lax.broadcasted_iota(jnp.int32, (E, E), 0)
    c8 = lax.broadcasted_iota(jnp.int32, (E, E), 1)
    tri8 = (r8 <= c8).astype(jnp.float32)
    p1 = jnp.max(p, axis=-1, keepdims=True)
    oh1 = (p == p1)
    cum1 = jnp.dot(oh1.astype(jnp.float32), tri8,
                   preferred_element_type=jnp.float32)
    first1 = oh1 & (cum1 == 1.0)
    pm = jnp.where(first1, -1.0, p)
    p2 = jnp.max(pm, axis=-1, keepdims=True)
    oh2 = (pm == p2)
    cum2 = jnp.dot(oh2.astype(jnp.float32), tri8,
                   preferred_element_type=jnp.float32)
    first2 = oh2 & (cum2 == 1.0)

    s1 = first1.astype(jnp.float32)
    s2 = first2.astype(jnp.float32)
    sel = s1 + s2

    rT = lax.broadcasted_iota(jnp.int32, (T, T), 0)
    cT = lax.broadcasted_iota(jnp.int32, (T, T), 1)
    ltri = (rT > cT).astype(jnp.bfloat16)
    rank = jnp.dot(ltri, sel.astype(jnp.bfloat16),
                   preferred_element_type=jnp.float32)

    counts = jnp.sum(sel, axis=0, keepdims=True)
    pc = jnp.ceil(counts / RB) * RB
    m8 = (r8 < c8).astype(jnp.float32)
    starts = jnp.dot(pc, m8, preferred_element_type=jnp.float32)

    slot = starts + rank
    rows0 = jnp.sum(jnp.where(first1, slot, 0.0), axis=-1, keepdims=True)
    rows1 = jnp.sum(jnp.where(first2, slot, 0.0), axis=-1, keepdims=True)
    rows0_ref[...] = rows0.astype(jnp.int32)
    rows1_ref[...] = rows1.astype(jnp.int32)

    p0 = jnp.sum(jnp.where(first1, p, 0.0), axis=-1, keepdims=True)
    p1v = jnp.sum(jnp.where(first2, p, 0.0), axis=-1, keepdims=True)
    gw0_ref[...] = jnp.broadcast_to(p0, (T, GW))
    gw1_ref[...] = jnp.broadcast_to(p1v, (T, GW))

    sb = starts / RB
    ib = lax.broadcasted_iota(jnp.int32, (NB, E), 0).astype(jnp.float32)
    cmp = (ib >= sb).astype(jnp.float32)
    bm = jnp.sum(cmp, axis=-1, keepdims=True) - 1.0
    bm_ref[...] = bm.astype(jnp.int32)

    iN = lax.broadcasted_iota(jnp.int32, (NB, NB), 0)
    jN = lax.broadcasted_iota(jnp.int32, (NB, NB), 1)
    shift = ((jN == iN - 1)).astype(jnp.float32)
    bmprev = jnp.dot(shift, bm, preferred_element_type=jnp.float32)
    i0 = (iN[:, 0:1] == 0)
    first = jnp.where(i0 | (bm != bmprev), 1.0, 0.0)
    triN = (jN <= iN).astype(jnp.float32)
    runidx = jnp.dot(triN, first, preferred_element_type=jnp.float32) - 1.0
    par = runidx - 2.0 * jnp.floor(runidx * 0.5)
    eyeN = (iN == jN).astype(jnp.float32)
    firstrow = jnp.dot(jnp.ones((1, NB), jnp.float32), first * eyeN,
                       preferred_element_type=jnp.float32)
    cand = jnp.where((firstrow > 0.0) & (jN > iN),
                     jN.astype(jnp.float32), float(NB))
    nb = jnp.min(cand, axis=-1, keepdims=True)
    fire = jnp.where((first > 0.0) & (nb < float(NB)), 1.0, 0.0)
    ohnb = (jN.astype(jnp.float32) == nb).astype(jnp.float32)
    nexte = jnp.dot(ohnb, bm, preferred_element_type=jnp.float32)
    par_ref[...] = par.astype(jnp.int32)
    first_ref[...] = first.astype(jnp.int32)
    fire_ref[...] = fire.astype(jnp.int32)
    nexte_ref[...] = nexte.astype(jnp.int32)


def _stage_a(probs):
    return pl.pallas_call(
        _router_body,
        in_specs=[
            pl.BlockSpec((T, E), lambda: (0, 0)),
        ],
        out_specs=[
            pl.BlockSpec((T, E), lambda: (0, 0)),
            pl.BlockSpec((T, 1), lambda: (0, 0)),
            pl.BlockSpec((T, 1), lambda: (0, 0)),
            pl.BlockSpec((T, GW), lambda: (0, 0)),
            pl.BlockSpec((T, GW), lambda: (0, 0)),
            pl.BlockSpec((NB, 1), lambda: (0, 0)),
            pl.BlockSpec((NB, 1), lambda: (0, 0)),
            pl.BlockSpec((NB, 1), lambda: (0, 0)),
            pl.BlockSpec((NB, 1), lambda: (0, 0)),
            pl.BlockSpec((NB, 1), lambda: (0, 0)),
        ],
        out_shape=[
            jax.ShapeDtypeStruct((T, E), jnp.float32),
            jax.ShapeDtypeStruct((T, 1), jnp.int32),
            jax.ShapeDtypeStruct((T, 1), jnp.int32),
            jax.ShapeDtypeStruct((T, GW), jnp.float32),
            jax.ShapeDtypeStruct((T, GW), jnp.float32),
            jax.ShapeDtypeStruct((NB, 1), jnp.int32),
            jax.ShapeDtypeStruct((NB, 1), jnp.int32),
            jax.ShapeDtypeStruct((NB, 1), jnp.int32),
            jax.ShapeDtypeStruct((NB, 1), jnp.int32),
            jax.ShapeDtypeStruct((NB, 1), jnp.int32),
        ],
    )(probs)


def _dispatch_body(x_hbm, rows0_hbm, rows1_hbm, gw0_hbm, gw1_hbm,
                   xs_hbm, g_hbm, idx0_v, idx1_v, xchunk, g0chunk, g1chunk,
                   sem):
    wid = lax.axis_index("s") * NC + lax.axis_index("c")
    base = wid * TPW
    pltpu.sync_copy(rows0_hbm.at[pl.ds(base, TPW)], idx0_v)
    pltpu.sync_copy(rows1_hbm.at[pl.ds(base, TPW)], idx1_v)
    pltpu.sync_copy(x_hbm.at[pl.ds(base, TPW), :], xchunk)
    pltpu.sync_copy(gw0_hbm.at[pl.ds(base, TPW), :], g0chunk)
    pltpu.sync_copy(gw1_hbm.at[pl.ds(base, TPW), :], g1chunk)
    c0 = pltpu.async_copy(xchunk, xs_hbm.at[idx0_v], sem)
    c1 = pltpu.async_copy(xchunk, xs_hbm.at[idx1_v], sem)
    c2 = pltpu.async_copy(g0chunk, g_hbm.at[idx0_v], sem)
    c3 = pltpu.async_copy(g1chunk, g_hbm.at[idx1_v], sem)
    c0.wait()
    c1.wait()
    c2.wait()
    c3.wait()


def _stage_b(x, rows0, rows1, gw0, gw1):
    mesh = plsc.VectorSubcoreMesh(core_axis_name="c", subcore_axis_name="s",
                                  num_cores=NC, num_subcores=NS)
    f = pl.kernel(
        _dispatch_body,
        out_type=[
            jax.ShapeDtypeStruct((PAD, D), jnp.float32),
            jax.ShapeDtypeStruct((PAD, GW), jnp.float32),
        ],
        mesh=mesh,
        scratch_types=[
            pltpu.VMEM((TPW,), jnp.int32),
            pltpu.VMEM((TPW,), jnp.int32),
            pltpu.VMEM((TPW, D), jnp.float32),
            pltpu.VMEM((TPW, GW), jnp.float32),
            pltpu.VMEM((TPW, GW), jnp.float32),
            pltpu.SemaphoreType.DMA,
        ],
    )
    return f(x, rows0, rows1, gw0, gw1)


def _ffn_body(bm_s, par_s, first_s, fire_s, nexte_s,
              xs_ref, g_ref, w1_any, w2_any, b1_ref, b2_ref, ys_ref,
              w1s0, w1s1, w2s0, w2s1, sem0, sem1):
    i = pl.program_id(0)
    e = bm_s[i]
    par = par_s[i]
    first = first_s[i]
    fire = fire_s[i]
    nxt = nexte_s[i]

    @pl.when(i == 0)
    def _prime():
        pltpu.make_async_copy(w1_any.at[pl.ds(e, 1)], w1s0, sem0).start()
        pltpu.make_async_copy(w2_any.at[pl.ds(e, 1)], w2s0, sem0).start()

    @pl.when((first == 1) & (par == 0))
    def _wait0():
        pltpu.make_async_copy(w1_any.at[pl.ds(e, 1)], w1s0, sem0).wait()
        pltpu.make_async_copy(w2_any.at[pl.ds(e, 1)], w2s0, sem0).wait()

    @pl.when((first == 1) & (par == 1))
    def _wait1():
        pltpu.make_async_copy(w1_any.at[pl.ds(e, 1)], w1s1, sem1).wait()
        pltpu.make_async_copy(w2_any.at[pl.ds(e, 1)], w2s1, sem1).wait()

    @pl.when((fire == 1) & (par == 0))
    def _fire1():
        pltpu.make_async_copy(w1_any.at[pl.ds(nxt, 1)], w1s1, sem1).start()
        pltpu.make_async_copy(w2_any.at[pl.ds(nxt, 1)], w2s1, sem1).start()

    @pl.when((fire == 1) & (par == 1))
    def _fire0():
        pltpu.make_async_copy(w1_any.at[pl.ds(nxt, 1)], w1s0, sem0).start()
        pltpu.make_async_copy(w2_any.at[pl.ds(nxt, 1)], w2s0, sem0).start()

    x = xs_ref[...].astype(jnp.bfloat16)

    def _ffn(w1s, w2s):
        h = jnp.dot(x, w1s[0].astype(jnp.bfloat16),
                    preferred_element_type=jnp.float32)
        h = jnp.maximum(h + b1_ref[0], 0.0)
        y = jnp.dot(h.astype(jnp.bfloat16), w2s[0].astype(jnp.bfloat16),
                    preferred_element_type=jnp.float32)
        return (y + b2_ref[0]) * g_ref[:, 0:1]

    @pl.when(par == 0)
    def _c0():
        ys_ref[...] = _ffn(w1s0, w2s0)

    @pl.when(par == 1)
    def _c1():
        ys_ref[...] = _ffn(w1s1, w2s1)


def _stage_c(bm, par, first, fire, nexte, xs, g, W1, b1, W2, b2):
    grid_spec = pltpu.PrefetchScalarGridSpec(
        num_scalar_prefetch=5,
        grid=(NB,),
        in_specs=[
            pl.BlockSpec((RB, D), lambda i, *s: (i, 0)),
            pl.BlockSpec((RB, GW), lambda i, *s: (i, 0)),
            pl.BlockSpec(memory_space=pl.ANY),
            pl.BlockSpec(memory_space=pl.ANY),
            pl.BlockSpec((1, 1, H), lambda i, bm, *s: (bm[i], 0, 0)),
            pl.BlockSpec((1, 1, D), lambda i, bm, *s: (bm[i], 0, 0)),
        ],
        out_specs=pl.BlockSpec((RB, D), lambda i, *s: (i, 0)),
        scratch_shapes=[
            pltpu.VMEM((1, D, H), jnp.float32),
            pltpu.VMEM((1, D, H), jnp.float32),
            pltpu.VMEM((1, H, D), jnp.float32),
            pltpu.VMEM((1, H, D), jnp.float32),
            pltpu.SemaphoreType.DMA,
            pltpu.SemaphoreType.DMA,
        ],
    )
    return pl.pallas_call(
        _ffn_body,
        grid_spec=grid_spec,
        out_shape=jax.ShapeDtypeStruct((PAD, D), jnp.float32),
    )(bm, par, first, fire, nexte, xs, g, W1, W2,
      b1.reshape(E, 1, H), b2.reshape(E, 1, D))


CHUNK = 16
NSUB = TPW // CHUNK


def _combine_body(ys_hbm, rows0_hbm, rows1_hbm, out_hbm,
                  idx0_a, idx1_a, buf0_a, buf1_a, idx0_b, idx1_b, buf0_b,
                  buf1_b, sem_a, sem_b):
    wid = lax.axis_index("s") * NC + lax.axis_index("c")
    bufs = [(idx0_a, idx1_a, buf0_a, buf1_a, sem_a),
            (idx0_b, idx1_b, buf0_b, buf1_b, sem_b)]

    def fire(k):
        idx0, idx1, b0, b1, sem = bufs[k % 2]
        base = wid * TPW + k * CHUNK
        pltpu.sync_copy(rows0_hbm.at[pl.ds(base, CHUNK)], idx0)
        pltpu.sync_copy(rows1_hbm.at[pl.ds(base, CHUNK)], idx1)
        return (pltpu.async_copy(ys_hbm.at[idx0], b0, sem),
                pltpu.async_copy(ys_hbm.at[idx1], b1, sem))

    def drain_compute(k, cps):
        idx0, idx1, b0, b1, sem = bufs[k % 2]
        base = wid * TPW + k * CHUNK
        cps[0].wait()
        cps[1].wait()

        def add_row(r, _):
            for c in range(D // 16):
                b0[r, pl.ds(c * 16, 16)] = (b0[r, pl.ds(c * 16, 16)]
                                            + b1[r, pl.ds(c * 16, 16)])
            return 0

        lax.fori_loop(0, CHUNK, add_row, 0)
        pltpu.sync_copy(b0, out_hbm.at[pl.ds(base, CHUNK), :])

    cps = fire(0)
    for k in range(NSUB):
        nxt = fire(k + 1) if k + 1 < NSUB else None
        drain_compute(k, cps)
        cps = nxt


def _stage_d(ys, rows0, rows1):
    mesh = plsc.VectorSubcoreMesh(core_axis_name="c", subcore_axis_name="s",
                                  num_cores=NC, num_subcores=NS)
    f = pl.kernel(
        _combine_body,
        out_type=jax.ShapeDtypeStruct((T, D), jnp.float32),
        mesh=mesh,
        scratch_types=[
            pltpu.VMEM((CHUNK,), jnp.int32),
            pltpu.VMEM((CHUNK,), jnp.int32),
            pltpu.VMEM((CHUNK, D), jnp.float32),
            pltpu.VMEM((CHUNK, D), jnp.float32),
            pltpu.VMEM((CHUNK,), jnp.int32),
            pltpu.VMEM((CHUNK,), jnp.int32),
            pltpu.VMEM((CHUNK, D), jnp.float32),
            pltpu.VMEM((CHUNK, D), jnp.float32),
            pltpu.SemaphoreType.DMA,
            pltpu.SemaphoreType.DMA,
        ],
    )
    return f(ys, rows0, rows1)


@jax.jit
def kernel(embedding, Wg, bg, R, W1, b1, W2, b2):
    x = embedding.reshape(T, D)

    state = jnp.zeros(embedding.shape[:-1] + (E,), dtype=embedding.dtype).at[..., 0].set(1.0)
    tb = jnp.einsum('ef,bsf->bse', R, state)
    logits = jnp.einsum('bsd,de->bse', embedding, Wg) + bg + tb
    probs_in = jax.nn.softmax(logits, axis=-1).reshape(T, E)

    (probs, rows0, rows1, gw0, gw1, bm, par, first, fire,
     nexte) = _stage_a(probs_in)
    rows0 = rows0.reshape(T)
    rows1 = rows1.reshape(T)
    xs, g = _stage_b(x, rows0, rows1, gw0, gw1)
    ys = _stage_c(bm.reshape(NB), par.reshape(NB), first.reshape(NB),
                  fire.reshape(NB), nexte.reshape(NB), xs, g, W1, b1, W2, b2)
    out = _stage_d(ys, rows0, rows1)
    return out.reshape(1, T, D), probs.reshape(1, T, E)
